# baseline (speedup 1.0000x reference)
import jax
import jax.numpy as jnp
from jax.experimental import pallas as pl
from jax.experimental.pallas import tpu as pltpu

GEOM = 3
PAUX = 3
KNN = 20
RB = 64


def _edge_pass_a(xa_ref, w1_ref, b1_ref, h1_ref, ea_ref, st_ref, oh_ref):
    fpp = xa_ref.shape[0]
    n = xa_ref.shape[1]
    c = xa_ref.shape[2]
    fin = c - PAUX
    nb = n // RB
    big = jnp.float32(3e38)
    for fr in range(fpp):
        xa = xa_ref[fr]
        x = xa[:, :fin]
        sq = jnp.sum(x * x, axis=1, keepdims=True)
        d2s = []
        for rb in range(nb):
            r0 = rb * RB
            xxt = jnp.dot(x[r0:r0 + RB, :], x.T,
                          preferred_element_type=jnp.float32)
            d2 = sq[r0:r0 + RB, :] + sq.T - 2.0 * xxt
            rows = jax.lax.broadcasted_iota(jnp.int32, (RB, n), 0) + r0
            cols = jax.lax.broadcasted_iota(jnp.int32, (RB, n), 1)
            d2s.append(jnp.where(rows == cols, d2 + 1e9, d2))
        for t in range(KNN):
            for rb in range(nb):
                d2 = d2s[rb]
                cols = jax.lax.broadcasted_iota(jnp.int32, (RB, n), 1)
                mn = jnp.min(d2, axis=1, keepdims=True)
                elig = d2 <= mn
                idx = jnp.min(jnp.where(elig, cols, n), axis=1, keepdims=True)
                oh = (cols == idx).astype(jnp.float32)
                d2s[rb] = jnp.where(cols == idx, big, d2)
                oh_ref[t * n + rb * RB:t * n + rb * RB + RB, :] = oh.astype(
                    jnp.bfloat16)
        xhi = xa.astype(jnp.bfloat16)
        r1 = xa - xhi.astype(jnp.float32)
        xmid = r1.astype(jnp.bfloat16)
        xlo = (r1 - xmid.astype(jnp.float32)).astype(jnp.bfloat16)
        ohb = oh_ref[...]
        g = (jnp.dot(ohb, xhi, preferred_element_type=jnp.float32)
             + jnp.dot(ohb, xmid, preferred_element_type=jnp.float32)
             + jnp.dot(ohb, xlo, preferred_element_type=jnp.float32))
        xi = jnp.concatenate([x] * KNN, axis=0)
        auxi = jnp.concatenate([xa[:, fin:]] * KNN, axis=0)
        e = jnp.concatenate([xi, g[:, :fin] - xi], axis=1)
        h1 = jnp.dot(e, w1_ref[...],
                     preferred_element_type=jnp.float32) + b1_ref[...]
        h1_ref[fr] = h1
        ea_ref[fr] = jnp.concatenate([auxi, g[:, fin:]], axis=1)
        st_ref[fr, 0:1, :] = jnp.sum(h1, axis=0, keepdims=True)
        st_ref[fr, 1:2, :] = jnp.sum(h1 * h1, axis=0, keepdims=True)


def _edge_pass_b(h1_ref, sc_ref, bi_ref, w2_ref, b2_ref, h2_ref, st_ref):
    fb = h1_ref.shape[0]
    m = h1_ref.shape[1]
    co = h1_ref.shape[2]
    h1 = h1_ref[...].reshape(fb * m, co)
    hb = jnp.maximum(h1 * sc_ref[...] + bi_ref[...], 0.0)
    h2 = jnp.dot(hb, w2_ref[...], preferred_element_type=jnp.float32) + b2_ref[...]
    h2_ref[...] = h2.reshape(fb, m, co)
    st_ref[0, 0:1, :] = jnp.sum(h2, axis=0, keepdims=True)
    st_ref[0, 1:2, :] = jnp.sum(h2 * h2, axis=0, keepdims=True)


def _edge_pass_c(h2_ref, ea_ref, sc_ref, bi_ref, a1_ref, a1b_ref,
                 a2_ref, a2b_ref, lg_ref, lb_ref, out_ref):
    kn = h2_ref.shape[1]
    co = h2_ref.shape[2]
    n = kn // KNN
    for fr in range(h2_ref.shape[0]):
        h = jnp.maximum(h2_ref[fr] * sc_ref[...] + bi_ref[...], 0.0)
        ea = ea_ref[fr]
        g1 = jnp.maximum(
            jnp.dot(ea, a1_ref[...], preferred_element_type=jnp.float32)
            + a1b_ref[...], 0.0)
        gb = jnp.dot(g1, a2_ref[...],
                     preferred_element_type=jnp.float32) + a2b_ref[...]
        m = jax.nn.sigmoid(gb[:, :co] + 1.0) * h + gb[:, co:]
        mx = jnp.max(m.reshape(KNN, n, co), axis=0)
        mu = jnp.mean(mx, axis=1, keepdims=True)
        v = jnp.mean((mx - mu) ** 2, axis=1, keepdims=True)
        y = lg_ref[...] * (mx - mu) / jnp.sqrt(v + 1e-5) + lb_ref[...]
        out_ref[fr] = jnp.maximum(y, 0.0)


def _lin1_pass(xc_ref, w_ref, b_ref, xl_ref, st_ref):
    xl = jnp.dot(xc_ref[...], w_ref[...],
                 preferred_element_type=jnp.float32) + b_ref[...]
    xl_ref[...] = xl
    st_ref[0, 0:1, :] = jnp.sum(xl, axis=0, keepdims=True)
    st_ref[0, 1:2, :] = jnp.sum(xl * xl, axis=0, keepdims=True)


def _pool_pass(xl_ref, sc_ref, bi_ref, e_ref):
    for fr in range(xl_ref.shape[0]):
        y = jnp.maximum(xl_ref[fr] * sc_ref[...] + bi_ref[...], 0.0)
        e_ref[fr] = jnp.max(y, axis=0, keepdims=True)


def _head_pass(e_ref, fs_ref, lng_ref, lnb_ref, wq_ref, wk_ref, wv_ref,
               wo_ref, pos_ref,
               lg0_ref, lb0_ref, wih0_ref, bih0_ref, whh0_ref, bhh0_ref,
               lg1_ref, lb1_ref, wih1_ref, bih1_ref, whh1_ref, bhh1_ref,
               w_a_ref, b_a_ref, w_b_ref, b_b_ref, w_c_ref, b_c_ref,
               w_d_ref, b_d_ref, out_ref):
    ft = e_ref.shape[0]
    d = e_ref.shape[1]
    bsz = out_ref.shape[0]
    tt = ft // bsz
    ee = e_ref[...]
    fs = fs_ref[...]
    chunks = []
    for ci in range(3):
        chv = fs[:, 3 * ci:3 * ci + 3]
        cmu = jnp.mean(chv, axis=1, keepdims=True)
        cv = jnp.mean((chv - cmu) ** 2, axis=1, keepdims=True)
        chunks.append(lng_ref[ci, :] * (chv - cmu) / jnp.sqrt(cv + 1e-5)
                      + lnb_ref[ci, :])
    s = jnp.concatenate(chunks, axis=1)
    q = jnp.dot(ee, wq_ref[...], preferred_element_type=jnp.float32)
    kk = jnp.dot(s, wk_ref[...], preferred_element_type=jnp.float32)
    vv = jnp.dot(s, wv_ref[...], preferred_element_type=jnp.float32)
    dca = q.shape[1]
    attn = jax.nn.sigmoid(q * kk * (dca ** -0.5))
    seq = ee + jnp.dot(attn * vv, wo_ref[...],
                       preferred_element_type=jnp.float32) + pos_ref[...]

    for (lg, lb, wih, bih, whh, bhh) in (
            (lg0_ref, lb0_ref, wih0_ref, bih0_ref, whh0_ref, bhh0_ref),
            (lg1_ref, lb1_ref, wih1_ref, bih1_ref, whh1_ref, bhh1_ref)):
        mu = jnp.mean(seq, axis=1, keepdims=True)
        v = jnp.mean((seq - mu) ** 2, axis=1, keepdims=True)
        hln = lg[...] * (seq - mu) / jnp.sqrt(v + 1e-5) + lb[...]
        h = jnp.zeros((bsz, d), jnp.float32)
        ys = []
        for t in range(tt):
            xt = hln[t * bsz:(t + 1) * bsz, :]
            gi = jnp.dot(xt, wih[...], preferred_element_type=jnp.float32) + bih[...]
            gh = jnp.dot(h, whh[...], preferred_element_type=jnp.float32) + bhh[...]
            r = jax.nn.sigmoid(gi[:, :d] + gh[:, :d])
            z = jax.nn.sigmoid(gi[:, d:2 * d] + gh[:, d:2 * d])
            nn = jnp.tanh(gi[:, 2 * d:] + r * gh[:, 2 * d:])
            h = (1.0 - z) * nn + z * h
            ys.append(h)
        seq = seq + jnp.concatenate(ys, axis=0)

    feat = jnp.zeros((bsz, d), jnp.float32)
    for t in range(tt):
        feat = feat + seq[t * bsz:(t + 1) * bsz, :]
    feat = feat * (1.0 / tt)
    for (wr, br) in ((w_a_ref, b_a_ref), (w_b_ref, b_b_ref), (w_c_ref, b_c_ref)):
        feat = jnp.maximum(
            jnp.dot(feat, wr[...], preferred_element_type=jnp.float32) + br[...],
            0.0)
    out_ref[...] = jnp.dot(feat, w_d_ref[...],
                           preferred_element_type=jnp.float32) + b_d_ref[...]


def _full(shape):
    nd = len(shape)
    return pl.BlockSpec(shape, lambda *a, _n=nd: (0,) * _n)


def _finalize_bn(st, g, b, n_rows):
    s1 = jnp.sum(st[:, 0, :], axis=0)
    s2 = jnp.sum(st[:, 1, :], axis=0)
    mu = s1 / n_rows
    var = s2 / n_rows - mu * mu
    scale = g / jnp.sqrt(var + 1e-5)
    bias = b - mu * scale
    return scale.reshape(1, -1), bias.reshape(1, -1)


def _edge_layer(x, aux, lp, f, n):
    fin = x.shape[2]
    co = lp['W1'].shape[1]
    kn = KNN * n
    xa = jnp.concatenate([x, aux], axis=2)
    c = fin + PAUX
    fpp = 2
    h1, ea, st1 = pl.pallas_call(
        _edge_pass_a,
        grid=(f // fpp,),
        in_specs=[pl.BlockSpec((fpp, n, c), lambda i: (i, 0, 0)),
                  _full((2 * fin, co)), _full((1, co))],
        out_specs=[pl.BlockSpec((fpp, kn, co), lambda i: (i, 0, 0)),
                   pl.BlockSpec((fpp, kn, 2 * PAUX), lambda i: (i, 0, 0)),
                   pl.BlockSpec((fpp, 2, co), lambda i: (i, 0, 0))],
        out_shape=[jax.ShapeDtypeStruct((f, kn, co), jnp.float32),
                   jax.ShapeDtypeStruct((f, kn, 2 * PAUX), jnp.float32),
                   jax.ShapeDtypeStruct((f, 2, co), jnp.float32)],
        scratch_shapes=[pltpu.VMEM((kn, n), jnp.bfloat16)],
    )(xa, lp['W1'], lp['b1'].reshape(1, -1))
    n_edges = f * kn
    sc1, bi1 = _finalize_bn(st1, lp['g1'], lp['e1'], n_edges)

    fb = 2
    h2, st2 = pl.pallas_call(
        _edge_pass_b,
        grid=(f // fb,),
        in_specs=[pl.BlockSpec((fb, kn, co), lambda i: (i, 0, 0)),
                  _full((1, co)), _full((1, co)),
                  _full((co, co)), _full((1, co))],
        out_specs=[pl.BlockSpec((fb, kn, co), lambda i: (i, 0, 0)),
                   pl.BlockSpec((1, 2, co), lambda i: (i, 0, 0))],
        out_shape=[jax.ShapeDtypeStruct((f, kn, co), jnp.float32),
                   jax.ShapeDtypeStruct((f // fb, 2, co), jnp.float32)],
    )(h1, sc1, bi1, lp['W2'], lp['b2'].reshape(1, -1))
    sc2, bi2 = _finalize_bn(st2, lp['g2'], lp['e2'], n_edges)

    gh = lp['A1'].shape[1]
    fc = 2
    out = pl.pallas_call(
        _edge_pass_c,
        grid=(f // fc,),
        in_specs=[pl.BlockSpec((fc, kn, co), lambda i: (i, 0, 0)),
                  pl.BlockSpec((fc, kn, 2 * PAUX), lambda i: (i, 0, 0)),
                  _full((1, co)), _full((1, co)),
                  _full((2 * PAUX, gh)), _full((1, gh)),
                  _full((gh, 2 * co)), _full((1, 2 * co)),
                  _full((1, co)), _full((1, co))],
        out_specs=pl.BlockSpec((fc, n, co), lambda i: (i, 0, 0)),
        out_shape=jax.ShapeDtypeStruct((f, n, co), jnp.float32),
    )(h2, ea, sc2, bi2, lp['A1'], lp['a1'].reshape(1, -1),
      lp['A2'], lp['a2'].reshape(1, -1),
      lp['lg'].reshape(1, -1), lp['lb'].reshape(1, -1))
    return out


def kernel(point_cloud, frame_signals, params):
    bsz, tt, n, _ = point_cloud.shape
    f = bsz * tt
    geom = point_cloud[..., :GEOM].reshape(f, n, GEOM)
    aux = point_cloud[..., GEOM:GEOM + PAUX].reshape(f, n, PAUX)

    x = geom
    xs = []
    for lp in params['layers']:
        x = _edge_layer(x, aux, lp, f, n)
        xs.append(x)
    xc = jnp.concatenate(xs, axis=2)
    sc_dim = xc.shape[2]
    l1 = params['lin1']
    d = l1['W'].shape[1]

    rows = f * n
    rblk = 2048
    xl, st = pl.pallas_call(
        _lin1_pass,
        grid=(rows // rblk,),
        in_specs=[pl.BlockSpec((rblk, sc_dim), lambda i: (i, 0)),
                  _full((sc_dim, d)), _full((1, d))],
        out_specs=[pl.BlockSpec((rblk, d), lambda i: (i, 0)),
                   pl.BlockSpec((1, 2, d), lambda i: (i, 0, 0))],
        out_shape=[jax.ShapeDtypeStruct((rows, d), jnp.float32),
                   jax.ShapeDtypeStruct((rows // rblk, 2, d), jnp.float32)],
    )(xc.reshape(rows, sc_dim), l1['W'], l1['b'].reshape(1, -1))
    scl, bil = _finalize_bn(st, l1['g'], l1['e'], rows)

    fp = 8
    e_frames = pl.pallas_call(
        _pool_pass,
        grid=(f // fp,),
        in_specs=[pl.BlockSpec((fp, n, d), lambda i: (i, 0, 0)),
                  _full((1, d)), _full((1, d))],
        out_specs=pl.BlockSpec((fp, 1, d), lambda i: (i, 0, 0)),
        out_shape=jax.ShapeDtypeStruct((f, 1, d), jnp.float32),
    )(xl.reshape(f, n, d), scl, bil)

    e_t = e_frames.reshape(bsz, tt, d).transpose(1, 0, 2).reshape(f, d)
    fs_t = frame_signals.transpose(1, 0, 2).reshape(f, -1)
    fca = params['fca']
    pos_t = jnp.repeat(params['pos'][0, :tt, :], bsz, axis=0)
    s0, s1 = params['ssm']
    o = params['out']
    lng = jnp.stack(fca['lng'])
    lnb = jnp.stack(fca['lnb'])

    out = pl.pallas_call(
        _head_pass,
        in_specs=[_full((f, d)), _full((f, fs_t.shape[1]))]
        + [_full(a.shape) for a in (lng, lnb)]
        + [_full(fca[w].shape) for w in ('Wq', 'Wk', 'Wv', 'Wo')]
        + [_full((f, d))]
        + [_full((1, d)), _full((1, d)), _full(s0['Wih'].shape),
           _full((1, s0['bih'].shape[0])), _full(s0['Whh'].shape),
           _full((1, s0['bhh'].shape[0]))] * 2
        + [_full(o['Ws'][0].shape), _full((1, o['bs'][0].shape[0])),
           _full(o['Ws'][1].shape), _full((1, o['bs'][1].shape[0])),
           _full(o['Ws'][2].shape), _full((1, o['bs'][2].shape[0])),
           _full(o['Ws'][3].shape), _full((1, o['bs'][3].shape[0]))],
        out_specs=_full((bsz, o['Ws'][3].shape[1])),
        out_shape=jax.ShapeDtypeStruct((bsz, o['Ws'][3].shape[1]), jnp.float32),
    )(e_t, fs_t, lng, lnb, fca['Wq'], fca['Wk'], fca['Wv'], fca['Wo'], pos_t,
      s0['lg'].reshape(1, -1), s0['lb'].reshape(1, -1), s0['Wih'],
      s0['bih'].reshape(1, -1), s0['Whh'], s0['bhh'].reshape(1, -1),
      s1['lg'].reshape(1, -1), s1['lb'].reshape(1, -1), s1['Wih'],
      s1['bih'].reshape(1, -1), s1['Whh'], s1['bhh'].reshape(1, -1),
      o['Ws'][0], o['bs'][0].reshape(1, -1), o['Ws'][1], o['bs'][1].reshape(1, -1),
      o['Ws'][2], o['bs'][2].reshape(1, -1), o['Ws'][3], o['bs'][3].reshape(1, -1))
    return out


# speedup vs baseline: 1.0602x; 1.0602x over previous
import jax
import jax.numpy as jnp
from jax.experimental import pallas as pl
from jax.experimental.pallas import tpu as pltpu

GEOM = 3
PAUX = 3
KNN = 20
RB = 64


def _edge_pass_a(xa_ref, w1_ref, b1_ref, h1_ref, ea_ref, st_ref, oh_ref):
    fpp = xa_ref.shape[0]
    n = xa_ref.shape[1]
    c = xa_ref.shape[2]
    fin = c - PAUX
    nb = n // RB
    big = jnp.float32(3e38)
    for fr in range(fpp):
        xa = xa_ref[fr]
        x = xa[:, :fin]
        sq = jnp.sum(x * x, axis=1, keepdims=True)
        d2s = []
        for rb in range(nb):
            r0 = rb * RB
            xxt = jnp.dot(x[r0:r0 + RB, :], x.T,
                          preferred_element_type=jnp.float32)
            d2 = sq[r0:r0 + RB, :] + sq.T - 2.0 * xxt
            rows = jax.lax.broadcasted_iota(jnp.int32, (RB, n), 0) + r0
            cols = jax.lax.broadcasted_iota(jnp.int32, (RB, n), 1)
            d2s.append(jnp.where(rows == cols, d2 + 1e9, d2))
        for t in range(KNN):
            for rb in range(nb):
                d2 = d2s[rb]
                cols = jax.lax.broadcasted_iota(jnp.int32, (RB, n), 1)
                mn = jnp.min(d2, axis=1, keepdims=True)
                elig = d2 <= mn
                idx = jnp.min(jnp.where(elig, cols, n), axis=1, keepdims=True)
                oh = (cols == idx).astype(jnp.float32)
                d2s[rb] = jnp.where(cols == idx, big, d2)
                oh_ref[t * n + rb * RB:t * n + rb * RB + RB, :] = oh.astype(
                    jnp.bfloat16)
        xhi = xa.astype(jnp.bfloat16)
        r1 = xa - xhi.astype(jnp.float32)
        xmid = r1.astype(jnp.bfloat16)
        xlo = (r1 - xmid.astype(jnp.float32)).astype(jnp.bfloat16)
        ohb = oh_ref[...]
        g = (jnp.dot(ohb, xhi, preferred_element_type=jnp.float32)
             + jnp.dot(ohb, xmid, preferred_element_type=jnp.float32)
             + jnp.dot(ohb, xlo, preferred_element_type=jnp.float32))
        xi = jnp.concatenate([x] * KNN, axis=0)
        auxi = jnp.concatenate([xa[:, fin:]] * KNN, axis=0)
        e = jnp.concatenate([xi, g[:, :fin] - xi], axis=1)
        h1 = jnp.dot(e, w1_ref[...],
                     preferred_element_type=jnp.float32) + b1_ref[...]
        h1_ref[fr] = h1
        ea_ref[fr] = jnp.concatenate([auxi, g[:, fin:]], axis=1)
        st_ref[fr, 0:1, :] = jnp.sum(h1, axis=0, keepdims=True)
        st_ref[fr, 1:2, :] = jnp.sum(h1 * h1, axis=0, keepdims=True)


def _edge_pass_b(h1_ref, sc_ref, bi_ref, w2_ref, b2_ref, h2_ref, st_ref):
    fb = h1_ref.shape[0]
    m = h1_ref.shape[1]
    co = h1_ref.shape[2]
    h1 = h1_ref[...].reshape(fb * m, co)
    hb = jnp.maximum(h1 * sc_ref[...] + bi_ref[...], 0.0)
    h2 = jnp.dot(hb, w2_ref[...], preferred_element_type=jnp.float32) + b2_ref[...]
    h2_ref[...] = h2.reshape(fb, m, co)
    st_ref[0, 0:1, :] = jnp.sum(h2, axis=0, keepdims=True)
    st_ref[0, 1:2, :] = jnp.sum(h2 * h2, axis=0, keepdims=True)


def _edge_pass_c(h2_ref, ea_ref, sc_ref, bi_ref, a1_ref, a1b_ref,
                 a2_ref, a2b_ref, lg_ref, lb_ref, out_ref):
    kn = h2_ref.shape[1]
    co = h2_ref.shape[2]
    n = kn // KNN
    for fr in range(h2_ref.shape[0]):
        h = jnp.maximum(h2_ref[fr] * sc_ref[...] + bi_ref[...], 0.0)
        ea = ea_ref[fr]
        g1 = jnp.maximum(
            jnp.dot(ea, a1_ref[...], preferred_element_type=jnp.float32)
            + a1b_ref[...], 0.0)
        gb = jnp.dot(g1, a2_ref[...],
                     preferred_element_type=jnp.float32) + a2b_ref[...]
        m = jax.nn.sigmoid(gb[:, :co] + 1.0) * h + gb[:, co:]
        mx = jnp.max(m.reshape(KNN, n, co), axis=0)
        mu = jnp.mean(mx, axis=1, keepdims=True)
        v = jnp.mean((mx - mu) ** 2, axis=1, keepdims=True)
        y = lg_ref[...] * (mx - mu) / jnp.sqrt(v + 1e-5) + lb_ref[...]
        out_ref[fr] = jnp.maximum(y, 0.0)


def _lin1_pass(xc_ref, w_ref, b_ref, xl_ref, st_ref):
    xl = jnp.dot(xc_ref[...], w_ref[...],
                 preferred_element_type=jnp.float32) + b_ref[...]
    xl_ref[...] = xl
    st_ref[0, 0:1, :] = jnp.sum(xl, axis=0, keepdims=True)
    st_ref[0, 1:2, :] = jnp.sum(xl * xl, axis=0, keepdims=True)


def _pool_pass(xl_ref, sc_ref, bi_ref, e_ref):
    for fr in range(xl_ref.shape[0]):
        y = jnp.maximum(xl_ref[fr] * sc_ref[...] + bi_ref[...], 0.0)
        e_ref[fr] = jnp.max(y, axis=0, keepdims=True)


def _head_pass(e_ref, fs_ref, lng_ref, lnb_ref, wq_ref, wk_ref, wv_ref,
               wo_ref, pos_ref,
               lg0_ref, lb0_ref, wih0_ref, bih0_ref, whh0_ref, bhh0_ref,
               lg1_ref, lb1_ref, wih1_ref, bih1_ref, whh1_ref, bhh1_ref,
               w_a_ref, b_a_ref, w_b_ref, b_b_ref, w_c_ref, b_c_ref,
               w_d_ref, b_d_ref, out_ref):
    ft = e_ref.shape[0]
    d = e_ref.shape[1]
    bsz = out_ref.shape[0]
    tt = ft // bsz
    ee = e_ref[...]
    fs = fs_ref[...]
    chunks = []
    for ci in range(3):
        chv = fs[:, 3 * ci:3 * ci + 3]
        cmu = jnp.mean(chv, axis=1, keepdims=True)
        cv = jnp.mean((chv - cmu) ** 2, axis=1, keepdims=True)
        chunks.append(lng_ref[ci, :] * (chv - cmu) / jnp.sqrt(cv + 1e-5)
                      + lnb_ref[ci, :])
    s = jnp.concatenate(chunks, axis=1)
    q = jnp.dot(ee, wq_ref[...], preferred_element_type=jnp.float32)
    kk = jnp.dot(s, wk_ref[...], preferred_element_type=jnp.float32)
    vv = jnp.dot(s, wv_ref[...], preferred_element_type=jnp.float32)
    dca = q.shape[1]
    attn = jax.nn.sigmoid(q * kk * (dca ** -0.5))
    seq = ee + jnp.dot(attn * vv, wo_ref[...],
                       preferred_element_type=jnp.float32) + pos_ref[...]

    for (lg, lb, wih, bih, whh, bhh) in (
            (lg0_ref, lb0_ref, wih0_ref, bih0_ref, whh0_ref, bhh0_ref),
            (lg1_ref, lb1_ref, wih1_ref, bih1_ref, whh1_ref, bhh1_ref)):
        mu = jnp.mean(seq, axis=1, keepdims=True)
        v = jnp.mean((seq - mu) ** 2, axis=1, keepdims=True)
        hln = lg[...] * (seq - mu) / jnp.sqrt(v + 1e-5) + lb[...]
        h = jnp.zeros((bsz, d), jnp.float32)
        ys = []
        for t in range(tt):
            xt = hln[t * bsz:(t + 1) * bsz, :]
            gi = jnp.dot(xt, wih[...], preferred_element_type=jnp.float32) + bih[...]
            gh = jnp.dot(h, whh[...], preferred_element_type=jnp.float32) + bhh[...]
            r = jax.nn.sigmoid(gi[:, :d] + gh[:, :d])
            z = jax.nn.sigmoid(gi[:, d:2 * d] + gh[:, d:2 * d])
            nn = jnp.tanh(gi[:, 2 * d:] + r * gh[:, 2 * d:])
            h = (1.0 - z) * nn + z * h
            ys.append(h)
        seq = seq + jnp.concatenate(ys, axis=0)

    feat = jnp.zeros((bsz, d), jnp.float32)
    for t in range(tt):
        feat = feat + seq[t * bsz:(t + 1) * bsz, :]
    feat = feat * (1.0 / tt)
    for (wr, br) in ((w_a_ref, b_a_ref), (w_b_ref, b_b_ref), (w_c_ref, b_c_ref)):
        feat = jnp.maximum(
            jnp.dot(feat, wr[...], preferred_element_type=jnp.float32) + br[...],
            0.0)
    out_ref[...] = jnp.dot(feat, w_d_ref[...],
                           preferred_element_type=jnp.float32) + b_d_ref[...]


def _full(shape):
    nd = len(shape)
    return pl.BlockSpec(shape, lambda *a, _n=nd: (0,) * _n)


def _finalize_bn(st, g, b, n_rows):
    s1 = jnp.sum(st[:, 0, :], axis=0)
    s2 = jnp.sum(st[:, 1, :], axis=0)
    mu = s1 / n_rows
    var = s2 / n_rows - mu * mu
    scale = g / jnp.sqrt(var + 1e-5)
    bias = b - mu * scale
    return scale.reshape(1, -1), bias.reshape(1, -1)


def _edge_layer(x, aux, lp, f, n):
    fin = x.shape[2]
    co = lp['W1'].shape[1]
    kn = KNN * n
    xa = jnp.concatenate([x, aux], axis=2)
    c = fin + PAUX
    fpp = 1
    h1, ea, st1 = pl.pallas_call(
        _edge_pass_a,
        grid=(f // fpp,),
        in_specs=[pl.BlockSpec((fpp, n, c), lambda i: (i, 0, 0)),
                  _full((2 * fin, co)), _full((1, co))],
        out_specs=[pl.BlockSpec((fpp, kn, co), lambda i: (i, 0, 0)),
                   pl.BlockSpec((fpp, kn, 2 * PAUX), lambda i: (i, 0, 0)),
                   pl.BlockSpec((fpp, 2, co), lambda i: (i, 0, 0))],
        out_shape=[jax.ShapeDtypeStruct((f, kn, co), jnp.float32),
                   jax.ShapeDtypeStruct((f, kn, 2 * PAUX), jnp.float32),
                   jax.ShapeDtypeStruct((f, 2, co), jnp.float32)],
        scratch_shapes=[pltpu.VMEM((kn, n), jnp.bfloat16)],
    )(xa, lp['W1'], lp['b1'].reshape(1, -1))
    n_edges = f * kn
    sc1, bi1 = _finalize_bn(st1, lp['g1'], lp['e1'], n_edges)

    fb = 2
    h2, st2 = pl.pallas_call(
        _edge_pass_b,
        grid=(f // fb,),
        in_specs=[pl.BlockSpec((fb, kn, co), lambda i: (i, 0, 0)),
                  _full((1, co)), _full((1, co)),
                  _full((co, co)), _full((1, co))],
        out_specs=[pl.BlockSpec((fb, kn, co), lambda i: (i, 0, 0)),
                   pl.BlockSpec((1, 2, co), lambda i: (i, 0, 0))],
        out_shape=[jax.ShapeDtypeStruct((f, kn, co), jnp.float32),
                   jax.ShapeDtypeStruct((f // fb, 2, co), jnp.float32)],
    )(h1, sc1, bi1, lp['W2'], lp['b2'].reshape(1, -1))
    sc2, bi2 = _finalize_bn(st2, lp['g2'], lp['e2'], n_edges)

    gh = lp['A1'].shape[1]
    fc = 1
    out = pl.pallas_call(
        _edge_pass_c,
        grid=(f // fc,),
        in_specs=[pl.BlockSpec((fc, kn, co), lambda i: (i, 0, 0)),
                  pl.BlockSpec((fc, kn, 2 * PAUX), lambda i: (i, 0, 0)),
                  _full((1, co)), _full((1, co)),
                  _full((2 * PAUX, gh)), _full((1, gh)),
                  _full((gh, 2 * co)), _full((1, 2 * co)),
                  _full((1, co)), _full((1, co))],
        out_specs=pl.BlockSpec((fc, n, co), lambda i: (i, 0, 0)),
        out_shape=jax.ShapeDtypeStruct((f, n, co), jnp.float32),
    )(h2, ea, sc2, bi2, lp['A1'], lp['a1'].reshape(1, -1),
      lp['A2'], lp['a2'].reshape(1, -1),
      lp['lg'].reshape(1, -1), lp['lb'].reshape(1, -1))
    return out


def kernel(point_cloud, frame_signals, params):
    bsz, tt, n, _ = point_cloud.shape
    f = bsz * tt
    geom = point_cloud[..., :GEOM].reshape(f, n, GEOM)
    aux = point_cloud[..., GEOM:GEOM + PAUX].reshape(f, n, PAUX)

    x = geom
    xs = []
    for lp in params['layers']:
        x = _edge_layer(x, aux, lp, f, n)
        xs.append(x)
    xc = jnp.concatenate(xs, axis=2)
    sc_dim = xc.shape[2]
    l1 = params['lin1']
    d = l1['W'].shape[1]

    rows = f * n
    rblk = 2048
    xl, st = pl.pallas_call(
        _lin1_pass,
        grid=(rows // rblk,),
        in_specs=[pl.BlockSpec((rblk, sc_dim), lambda i: (i, 0)),
                  _full((sc_dim, d)), _full((1, d))],
        out_specs=[pl.BlockSpec((rblk, d), lambda i: (i, 0)),
                   pl.BlockSpec((1, 2, d), lambda i: (i, 0, 0))],
        out_shape=[jax.ShapeDtypeStruct((rows, d), jnp.float32),
                   jax.ShapeDtypeStruct((rows // rblk, 2, d), jnp.float32)],
    )(xc.reshape(rows, sc_dim), l1['W'], l1['b'].reshape(1, -1))
    scl, bil = _finalize_bn(st, l1['g'], l1['e'], rows)

    fp = 8
    e_frames = pl.pallas_call(
        _pool_pass,
        grid=(f // fp,),
        in_specs=[pl.BlockSpec((fp, n, d), lambda i: (i, 0, 0)),
                  _full((1, d)), _full((1, d))],
        out_specs=pl.BlockSpec((fp, 1, d), lambda i: (i, 0, 0)),
        out_shape=jax.ShapeDtypeStruct((f, 1, d), jnp.float32),
    )(xl.reshape(f, n, d), scl, bil)

    e_t = e_frames.reshape(bsz, tt, d).transpose(1, 0, 2).reshape(f, d)
    fs_t = frame_signals.transpose(1, 0, 2).reshape(f, -1)
    fca = params['fca']
    pos_t = jnp.repeat(params['pos'][0, :tt, :], bsz, axis=0)
    s0, s1 = params['ssm']
    o = params['out']
    lng = jnp.stack(fca['lng'])
    lnb = jnp.stack(fca['lnb'])

    out = pl.pallas_call(
        _head_pass,
        in_specs=[_full((f, d)), _full((f, fs_t.shape[1]))]
        + [_full(a.shape) for a in (lng, lnb)]
        + [_full(fca[w].shape) for w in ('Wq', 'Wk', 'Wv', 'Wo')]
        + [_full((f, d))]
        + [_full((1, d)), _full((1, d)), _full(s0['Wih'].shape),
           _full((1, s0['bih'].shape[0])), _full(s0['Whh'].shape),
           _full((1, s0['bhh'].shape[0]))] * 2
        + [_full(o['Ws'][0].shape), _full((1, o['bs'][0].shape[0])),
           _full(o['Ws'][1].shape), _full((1, o['bs'][1].shape[0])),
           _full(o['Ws'][2].shape), _full((1, o['bs'][2].shape[0])),
           _full(o['Ws'][3].shape), _full((1, o['bs'][3].shape[0]))],
        out_specs=_full((bsz, o['Ws'][3].shape[1])),
        out_shape=jax.ShapeDtypeStruct((bsz, o['Ws'][3].shape[1]), jnp.float32),
    )(e_t, fs_t, lng, lnb, fca['Wq'], fca['Wk'], fca['Wv'], fca['Wo'], pos_t,
      s0['lg'].reshape(1, -1), s0['lb'].reshape(1, -1), s0['Wih'],
      s0['bih'].reshape(1, -1), s0['Whh'], s0['bhh'].reshape(1, -1),
      s1['lg'].reshape(1, -1), s1['lb'].reshape(1, -1), s1['Wih'],
      s1['bih'].reshape(1, -1), s1['Whh'], s1['bhh'].reshape(1, -1),
      o['Ws'][0], o['bs'][0].reshape(1, -1), o['Ws'][1], o['bs'][1].reshape(1, -1),
      o['Ws'][2], o['bs'][2].reshape(1, -1), o['Ws'][3], o['bs'][3].reshape(1, -1))
    return out


# speedup vs baseline: 1.1802x; 1.1132x over previous
import jax
import jax.numpy as jnp
from jax.experimental import pallas as pl
from jax.experimental.pallas import tpu as pltpu

GEOM = 3
PAUX = 3
KNN = 20
RB = 64


def _edge_pass_a(xa_ref, w1_ref, b1_ref, h1_ref, ea_ref, st_ref, oh_ref):
    fpp = xa_ref.shape[0]
    n = xa_ref.shape[1]
    c = xa_ref.shape[2]
    fin = c - PAUX
    nb = n // RB
    big = jnp.float32(3e38)
    for fr in range(fpp):
        xa = xa_ref[fr]
        x = xa[:, :fin]
        sq = jnp.sum(x * x, axis=1, keepdims=True)
        d2s = []
        for rb in range(nb):
            r0 = rb * RB
            xxt = jnp.dot(x[r0:r0 + RB, :], x.T,
                          preferred_element_type=jnp.float32)
            d2 = sq[r0:r0 + RB, :] + sq.T - 2.0 * xxt
            rows = jax.lax.broadcasted_iota(jnp.int32, (RB, n), 0) + r0
            cols = jax.lax.broadcasted_iota(jnp.int32, (RB, n), 1)
            d2s.append(jnp.where(rows == cols, d2 + 1e9, d2))
        for t in range(KNN):
            for rb in range(nb):
                d2 = d2s[rb]
                colsf = jax.lax.broadcasted_iota(
                    jnp.int32, (RB, n), 1).astype(jnp.float32)
                mn = jnp.min(d2, axis=1, keepdims=True)
                elig = d2 <= mn
                idx = jnp.min(jnp.where(elig, colsf, jnp.float32(n)),
                              axis=1, keepdims=True)
                oh = (colsf == idx).astype(jnp.float32)
                d2s[rb] = jnp.where(colsf == idx, big, d2)
                oh_ref[t * n + rb * RB:t * n + rb * RB + RB, :] = oh.astype(
                    jnp.bfloat16)
        xhi = xa.astype(jnp.bfloat16)
        r1 = xa - xhi.astype(jnp.float32)
        xmid = r1.astype(jnp.bfloat16)
        xlo = (r1 - xmid.astype(jnp.float32)).astype(jnp.bfloat16)
        ohb = oh_ref[...]
        g = (jnp.dot(ohb, xhi, preferred_element_type=jnp.float32)
             + jnp.dot(ohb, xmid, preferred_element_type=jnp.float32)
             + jnp.dot(ohb, xlo, preferred_element_type=jnp.float32))
        xi = jnp.concatenate([x] * KNN, axis=0)
        auxi = jnp.concatenate([xa[:, fin:]] * KNN, axis=0)
        e = jnp.concatenate([xi, g[:, :fin] - xi], axis=1)
        h1 = jnp.dot(e, w1_ref[...],
                     preferred_element_type=jnp.float32) + b1_ref[...]
        h1_ref[fr] = h1
        ea_ref[fr] = jnp.concatenate([auxi, g[:, fin:]], axis=1)
        st_ref[fr, 0:1, :] = jnp.sum(h1, axis=0, keepdims=True)
        st_ref[fr, 1:2, :] = jnp.sum(h1 * h1, axis=0, keepdims=True)


def _edge_pass_b(h1_ref, sc_ref, bi_ref, w2_ref, b2_ref, h2_ref, st_ref):
    fb = h1_ref.shape[0]
    m = h1_ref.shape[1]
    co = h1_ref.shape[2]
    h1 = h1_ref[...].reshape(fb * m, co)
    hb = jnp.maximum(h1 * sc_ref[...] + bi_ref[...], 0.0)
    h2 = jnp.dot(hb, w2_ref[...], preferred_element_type=jnp.float32) + b2_ref[...]
    h2_ref[...] = h2.reshape(fb, m, co)
    st_ref[0, 0:1, :] = jnp.sum(h2, axis=0, keepdims=True)
    st_ref[0, 1:2, :] = jnp.sum(h2 * h2, axis=0, keepdims=True)


def _edge_pass_c(h2_ref, ea_ref, sc_ref, bi_ref, a1_ref, a1b_ref,
                 a2_ref, a2b_ref, lg_ref, lb_ref, out_ref):
    kn = h2_ref.shape[1]
    co = h2_ref.shape[2]
    n = kn // KNN
    for fr in range(h2_ref.shape[0]):
        h = jnp.maximum(h2_ref[fr] * sc_ref[...] + bi_ref[...], 0.0)
        ea = ea_ref[fr]
        g1 = jnp.maximum(
            jnp.dot(ea, a1_ref[...], preferred_element_type=jnp.float32)
            + a1b_ref[...], 0.0)
        gb = jnp.dot(g1, a2_ref[...],
                     preferred_element_type=jnp.float32) + a2b_ref[...]
        m = jax.nn.sigmoid(gb[:, :co] + 1.0) * h + gb[:, co:]
        mx = jnp.max(m.reshape(KNN, n, co), axis=0)
        mu = jnp.mean(mx, axis=1, keepdims=True)
        v = jnp.mean((mx - mu) ** 2, axis=1, keepdims=True)
        y = lg_ref[...] * (mx - mu) / jnp.sqrt(v + 1e-5) + lb_ref[...]
        out_ref[fr] = jnp.maximum(y, 0.0)


def _lin1_pass(xc_ref, w_ref, b_ref, xl_ref, st_ref):
    xl = jnp.dot(xc_ref[...], w_ref[...],
                 preferred_element_type=jnp.float32) + b_ref[...]
    xl_ref[...] = xl
    st_ref[0, 0:1, :] = jnp.sum(xl, axis=0, keepdims=True)
    st_ref[0, 1:2, :] = jnp.sum(xl * xl, axis=0, keepdims=True)


def _pool_pass(xl_ref, sc_ref, bi_ref, e_ref):
    for fr in range(xl_ref.shape[0]):
        y = jnp.maximum(xl_ref[fr] * sc_ref[...] + bi_ref[...], 0.0)
        e_ref[fr] = jnp.max(y, axis=0, keepdims=True)


def _head_pass(e_ref, fs_ref, lng_ref, lnb_ref, wq_ref, wk_ref, wv_ref,
               wo_ref, pos_ref,
               lg0_ref, lb0_ref, wih0_ref, bih0_ref, whh0_ref, bhh0_ref,
               lg1_ref, lb1_ref, wih1_ref, bih1_ref, whh1_ref, bhh1_ref,
               w_a_ref, b_a_ref, w_b_ref, b_b_ref, w_c_ref, b_c_ref,
               w_d_ref, b_d_ref, out_ref):
    ft = e_ref.shape[0]
    d = e_ref.shape[1]
    bsz = out_ref.shape[0]
    tt = ft // bsz
    ee = e_ref[...]
    fs = fs_ref[...]
    chunks = []
    for ci in range(3):
        chv = fs[:, 3 * ci:3 * ci + 3]
        cmu = jnp.mean(chv, axis=1, keepdims=True)
        cv = jnp.mean((chv - cmu) ** 2, axis=1, keepdims=True)
        chunks.append(lng_ref[ci, :] * (chv - cmu) / jnp.sqrt(cv + 1e-5)
                      + lnb_ref[ci, :])
    s = jnp.concatenate(chunks, axis=1)
    q = jnp.dot(ee, wq_ref[...], preferred_element_type=jnp.float32)
    kk = jnp.dot(s, wk_ref[...], preferred_element_type=jnp.float32)
    vv = jnp.dot(s, wv_ref[...], preferred_element_type=jnp.float32)
    dca = q.shape[1]
    attn = jax.nn.sigmoid(q * kk * (dca ** -0.5))
    seq = ee + jnp.dot(attn * vv, wo_ref[...],
                       preferred_element_type=jnp.float32) + pos_ref[...]

    for (lg, lb, wih, bih, whh, bhh) in (
            (lg0_ref, lb0_ref, wih0_ref, bih0_ref, whh0_ref, bhh0_ref),
            (lg1_ref, lb1_ref, wih1_ref, bih1_ref, whh1_ref, bhh1_ref)):
        mu = jnp.mean(seq, axis=1, keepdims=True)
        v = jnp.mean((seq - mu) ** 2, axis=1, keepdims=True)
        hln = lg[...] * (seq - mu) / jnp.sqrt(v + 1e-5) + lb[...]
        h = jnp.zeros((bsz, d), jnp.float32)
        ys = []
        for t in range(tt):
            xt = hln[t * bsz:(t + 1) * bsz, :]
            gi = jnp.dot(xt, wih[...], preferred_element_type=jnp.float32) + bih[...]
            gh = jnp.dot(h, whh[...], preferred_element_type=jnp.float32) + bhh[...]
            r = jax.nn.sigmoid(gi[:, :d] + gh[:, :d])
            z = jax.nn.sigmoid(gi[:, d:2 * d] + gh[:, d:2 * d])
            nn = jnp.tanh(gi[:, 2 * d:] + r * gh[:, 2 * d:])
            h = (1.0 - z) * nn + z * h
            ys.append(h)
        seq = seq + jnp.concatenate(ys, axis=0)

    feat = jnp.zeros((bsz, d), jnp.float32)
    for t in range(tt):
        feat = feat + seq[t * bsz:(t + 1) * bsz, :]
    feat = feat * (1.0 / tt)
    for (wr, br) in ((w_a_ref, b_a_ref), (w_b_ref, b_b_ref), (w_c_ref, b_c_ref)):
        feat = jnp.maximum(
            jnp.dot(feat, wr[...], preferred_element_type=jnp.float32) + br[...],
            0.0)
    out_ref[...] = jnp.dot(feat, w_d_ref[...],
                           preferred_element_type=jnp.float32) + b_d_ref[...]


def _full(shape):
    nd = len(shape)
    return pl.BlockSpec(shape, lambda *a, _n=nd: (0,) * _n)


def _finalize_bn(st, g, b, n_rows):
    s1 = jnp.sum(st[:, 0, :], axis=0)
    s2 = jnp.sum(st[:, 1, :], axis=0)
    mu = s1 / n_rows
    var = s2 / n_rows - mu * mu
    scale = g / jnp.sqrt(var + 1e-5)
    bias = b - mu * scale
    return scale.reshape(1, -1), bias.reshape(1, -1)


def _edge_layer(x, aux, lp, f, n):
    fin = x.shape[2]
    co = lp['W1'].shape[1]
    kn = KNN * n
    xa = jnp.concatenate([x, aux], axis=2)
    c = fin + PAUX
    fpp = 1
    h1, ea, st1 = pl.pallas_call(
        _edge_pass_a,
        grid=(f // fpp,),
        in_specs=[pl.BlockSpec((fpp, n, c), lambda i: (i, 0, 0)),
                  _full((2 * fin, co)), _full((1, co))],
        out_specs=[pl.BlockSpec((fpp, kn, co), lambda i: (i, 0, 0)),
                   pl.BlockSpec((fpp, kn, 2 * PAUX), lambda i: (i, 0, 0)),
                   pl.BlockSpec((fpp, 2, co), lambda i: (i, 0, 0))],
        out_shape=[jax.ShapeDtypeStruct((f, kn, co), jnp.float32),
                   jax.ShapeDtypeStruct((f, kn, 2 * PAUX), jnp.float32),
                   jax.ShapeDtypeStruct((f, 2, co), jnp.float32)],
        scratch_shapes=[pltpu.VMEM((kn, n), jnp.bfloat16)],
    )(xa, lp['W1'], lp['b1'].reshape(1, -1))
    n_edges = f * kn
    sc1, bi1 = _finalize_bn(st1, lp['g1'], lp['e1'], n_edges)

    fb = 2
    h2, st2 = pl.pallas_call(
        _edge_pass_b,
        grid=(f // fb,),
        in_specs=[pl.BlockSpec((fb, kn, co), lambda i: (i, 0, 0)),
                  _full((1, co)), _full((1, co)),
                  _full((co, co)), _full((1, co))],
        out_specs=[pl.BlockSpec((fb, kn, co), lambda i: (i, 0, 0)),
                   pl.BlockSpec((1, 2, co), lambda i: (i, 0, 0))],
        out_shape=[jax.ShapeDtypeStruct((f, kn, co), jnp.float32),
                   jax.ShapeDtypeStruct((f // fb, 2, co), jnp.float32)],
    )(h1, sc1, bi1, lp['W2'], lp['b2'].reshape(1, -1))
    sc2, bi2 = _finalize_bn(st2, lp['g2'], lp['e2'], n_edges)

    gh = lp['A1'].shape[1]
    fc = 1
    out = pl.pallas_call(
        _edge_pass_c,
        grid=(f // fc,),
        in_specs=[pl.BlockSpec((fc, kn, co), lambda i: (i, 0, 0)),
                  pl.BlockSpec((fc, kn, 2 * PAUX), lambda i: (i, 0, 0)),
                  _full((1, co)), _full((1, co)),
                  _full((2 * PAUX, gh)), _full((1, gh)),
                  _full((gh, 2 * co)), _full((1, 2 * co)),
                  _full((1, co)), _full((1, co))],
        out_specs=pl.BlockSpec((fc, n, co), lambda i: (i, 0, 0)),
        out_shape=jax.ShapeDtypeStruct((f, n, co), jnp.float32),
    )(h2, ea, sc2, bi2, lp['A1'], lp['a1'].reshape(1, -1),
      lp['A2'], lp['a2'].reshape(1, -1),
      lp['lg'].reshape(1, -1), lp['lb'].reshape(1, -1))
    return out


def kernel(point_cloud, frame_signals, params):
    bsz, tt, n, _ = point_cloud.shape
    f = bsz * tt
    geom = point_cloud[..., :GEOM].reshape(f, n, GEOM)
    aux = point_cloud[..., GEOM:GEOM + PAUX].reshape(f, n, PAUX)

    x = geom
    xs = []
    for lp in params['layers']:
        x = _edge_layer(x, aux, lp, f, n)
        xs.append(x)
    xc = jnp.concatenate(xs, axis=2)
    sc_dim = xc.shape[2]
    l1 = params['lin1']
    d = l1['W'].shape[1]

    rows = f * n
    rblk = 2048
    xl, st = pl.pallas_call(
        _lin1_pass,
        grid=(rows // rblk,),
        in_specs=[pl.BlockSpec((rblk, sc_dim), lambda i: (i, 0)),
                  _full((sc_dim, d)), _full((1, d))],
        out_specs=[pl.BlockSpec((rblk, d), lambda i: (i, 0)),
                   pl.BlockSpec((1, 2, d), lambda i: (i, 0, 0))],
        out_shape=[jax.ShapeDtypeStruct((rows, d), jnp.float32),
                   jax.ShapeDtypeStruct((rows // rblk, 2, d), jnp.float32)],
    )(xc.reshape(rows, sc_dim), l1['W'], l1['b'].reshape(1, -1))
    scl, bil = _finalize_bn(st, l1['g'], l1['e'], rows)

    fp = 8
    e_frames = pl.pallas_call(
        _pool_pass,
        grid=(f // fp,),
        in_specs=[pl.BlockSpec((fp, n, d), lambda i: (i, 0, 0)),
                  _full((1, d)), _full((1, d))],
        out_specs=pl.BlockSpec((fp, 1, d), lambda i: (i, 0, 0)),
        out_shape=jax.ShapeDtypeStruct((f, 1, d), jnp.float32),
    )(xl.reshape(f, n, d), scl, bil)

    e_t = e_frames.reshape(bsz, tt, d).transpose(1, 0, 2).reshape(f, d)
    fs_t = frame_signals.transpose(1, 0, 2).reshape(f, -1)
    fca = params['fca']
    pos_t = jnp.repeat(params['pos'][0, :tt, :], bsz, axis=0)
    s0, s1 = params['ssm']
    o = params['out']
    lng = jnp.stack(fca['lng'])
    lnb = jnp.stack(fca['lnb'])

    out = pl.pallas_call(
        _head_pass,
        in_specs=[_full((f, d)), _full((f, fs_t.shape[1]))]
        + [_full(a.shape) for a in (lng, lnb)]
        + [_full(fca[w].shape) for w in ('Wq', 'Wk', 'Wv', 'Wo')]
        + [_full((f, d))]
        + [_full((1, d)), _full((1, d)), _full(s0['Wih'].shape),
           _full((1, s0['bih'].shape[0])), _full(s0['Whh'].shape),
           _full((1, s0['bhh'].shape[0]))] * 2
        + [_full(o['Ws'][0].shape), _full((1, o['bs'][0].shape[0])),
           _full(o['Ws'][1].shape), _full((1, o['bs'][1].shape[0])),
           _full(o['Ws'][2].shape), _full((1, o['bs'][2].shape[0])),
           _full(o['Ws'][3].shape), _full((1, o['bs'][3].shape[0]))],
        out_specs=_full((bsz, o['Ws'][3].shape[1])),
        out_shape=jax.ShapeDtypeStruct((bsz, o['Ws'][3].shape[1]), jnp.float32),
    )(e_t, fs_t, lng, lnb, fca['Wq'], fca['Wk'], fca['Wv'], fca['Wo'], pos_t,
      s0['lg'].reshape(1, -1), s0['lb'].reshape(1, -1), s0['Wih'],
      s0['bih'].reshape(1, -1), s0['Whh'], s0['bhh'].reshape(1, -1),
      s1['lg'].reshape(1, -1), s1['lb'].reshape(1, -1), s1['Wih'],
      s1['bih'].reshape(1, -1), s1['Whh'], s1['bhh'].reshape(1, -1),
      o['Ws'][0], o['bs'][0].reshape(1, -1), o['Ws'][1], o['bs'][1].reshape(1, -1),
      o['Ws'][2], o['bs'][2].reshape(1, -1), o['Ws'][3], o['bs'][3].reshape(1, -1))
    return out
